# baseline (speedup 1.0000x reference)
import functools

import jax
import jax.numpy as jnp
from jax.experimental import pallas as pl
from jax.experimental.pallas import tpu as pltpu

_P = 64
_NC = 41
_KW = 128
_NEG = -1.0e9


def _pattern_kernel(idx_ref, w_ref, out_ref, *, n_rows):
    rows = idx_ref.shape[0]
    tb = rows // n_rows
    i4 = idx_ref[...]
    iota = jax.lax.broadcasted_iota(jnp.int32, (rows, _KW), 1)
    m = (i4[:, 0:1] == iota)
    m = m | (i4[:, 1:2] == iota - _NC)
    m = m | (i4[:, 2:3] == iota - 2 * _NC)
    m = m | (iota == 3 * _NC)
    m = m | (i4[:, 3:4] == iota - 3 * _NC)
    oh = m.astype(jnp.bfloat16)
    y = jnp.dot(oh, w_ref[...], preferred_element_type=jnp.float32)
    b4s = jnp.concatenate(
        [y[4:, 192:256], jnp.zeros((4, _P), jnp.float32)], axis=0)
    s12 = jnp.maximum(y[:, 0:128], 0.0)
    s4 = jnp.maximum(y[:, 128:192] + b4s, 0.0)
    r12 = jnp.sum(s12.reshape(tb, n_rows, 128), axis=1)
    r4 = jnp.sum(s4.reshape(tb, n_rows, _P), axis=1)
    out_ref[...] = r12[:, 0:_P] + r12[:, _P:2 * _P] + r4


@functools.partial(jax.jit, static_argnames=("block_boards",))
def _pattern_module(discarded_idx, card_features, W1, b1, W2, b2, W4, b4,
                    *, block_boards=128):
    B, N = discarded_idx.shape
    P = _P
    nc = _NC

    def tap(Wd, k):
        return jnp.dot(card_features, jnp.transpose(Wd[:, :, k]))

    f1, f2, f4 = 1.0 / (N - 1), 1.0 / (N - 2), 1.0 / (N - 4)
    w = jnp.zeros((_KW, 4 * P), jnp.float32)
    w = w.at[0:40, 0 * P:1 * P].set(tap(W1, 0) * f1)
    w = w.at[0:40, 1 * P:2 * P].set(tap(W2, 0) * f2)
    w = w.at[0:40, 2 * P:3 * P].set(tap(W4, 0) * f4)
    w = w.at[0:40, 3 * P:4 * P].set(tap(W4, 1) * f4)
    w = w.at[nc:nc + 40, 0 * P:1 * P].set(tap(W1, 1) * f1)
    w = w.at[nc + 40, 0 * P:1 * P].set(_NEG)
    w = w.at[2 * nc:2 * nc + 40, 1 * P:2 * P].set(tap(W2, 1) * f2)
    w = w.at[2 * nc + 40, 1 * P:2 * P].set(_NEG)
    w = w.at[3 * nc, 0 * P:1 * P].set(b1 * f1)
    w = w.at[3 * nc, 1 * P:2 * P].set(b2 * f2)
    w = w.at[3 * nc, 2 * P:3 * P].set(b4 * f4)
    w = w.at[3 * nc + 1, 2 * P:3 * P].set(_NEG)
    w = w.astype(jnp.bfloat16)

    idx = discarded_idx.astype(jnp.int32)
    edge = jnp.full((B, 1), 40, jnp.int32)
    cols = [idx]
    for off in (1, 2):
        cols.append(jnp.concatenate(
            [idx[:, off:], jnp.broadcast_to(edge, (B, off))], axis=1))
    t = jnp.arange(N, dtype=jnp.int32)
    cols.append(jnp.broadcast_to((t >= N - 4).astype(jnp.int32), (B, N)))
    idx4 = jnp.stack(cols, axis=-1).reshape(B * N, 4)

    tb = block_boards
    body = functools.partial(_pattern_kernel, n_rows=N)
    out = pl.pallas_call(
        body,
        out_shape=jax.ShapeDtypeStruct((B, P), jnp.float32),
        grid=(B // tb,),
        in_specs=[
            pl.BlockSpec((tb * N, 4), lambda i: (i, 0)),
            pl.BlockSpec((_KW, 4 * P), lambda i: (0, 0)),
        ],
        out_specs=pl.BlockSpec((tb, P), lambda i: (i, 0)),
        compiler_params=pltpu.CompilerParams(
            dimension_semantics=("parallel",)),
    )(idx4, w)
    return out


def kernel(discarded_idx, card_features, W1, b1, W2, b2, W4, b4):
    return _pattern_module(discarded_idx, card_features,
                           W1, b1, W2, b2, W4, b4)


# speedup vs baseline: 1.0066x; 1.0066x over previous
import functools

import jax
import jax.numpy as jnp
from jax.experimental import pallas as pl
from jax.experimental.pallas import tpu as pltpu

_P = 64
_NC = 41
_KW = 128
_NEG = -1.0e9


def _pattern_kernel(idx_ref, w_ref, out_ref, *, n_rows):
    rows = idx_ref.shape[0]
    tb = rows // n_rows
    i4 = idx_ref[...]
    iota = jax.lax.broadcasted_iota(jnp.int32, (rows, _KW), 1)
    m = (i4[:, 0:1] == iota)
    m = m | (i4[:, 1:2] == iota - _NC)
    m = m | (i4[:, 2:3] == iota - 2 * _NC)
    m = m | (iota == 3 * _NC)
    m = m | (i4[:, 3:4] == iota - 3 * _NC)
    oh = m.astype(jnp.bfloat16)
    y = jnp.dot(oh, w_ref[...], preferred_element_type=jnp.float32)
    b4s = jnp.concatenate(
        [y[4:, 192:256], jnp.zeros((4, _P), jnp.float32)], axis=0)
    s12 = jnp.maximum(y[:, 0:128], 0.0)
    s4 = jnp.maximum(y[:, 128:192] + b4s, 0.0)
    r12 = jnp.sum(s12.reshape(tb, n_rows, 128), axis=1)
    r4 = jnp.sum(s4.reshape(tb, n_rows, _P), axis=1)
    out_ref[...] = r12[:, 0:_P] + r12[:, _P:2 * _P] + r4


@functools.partial(jax.jit, static_argnames=("block_boards",))
def _pattern_module(discarded_idx, card_features, W1, b1, W2, b2, W4, b4,
                    *, block_boards=256):
    B, N = discarded_idx.shape
    P = _P
    nc = _NC

    def tap(Wd, k):
        return jnp.dot(card_features, jnp.transpose(Wd[:, :, k]))

    f1, f2, f4 = 1.0 / (N - 1), 1.0 / (N - 2), 1.0 / (N - 4)
    w = jnp.zeros((_KW, 4 * P), jnp.float32)
    w = w.at[0:40, 0 * P:1 * P].set(tap(W1, 0) * f1)
    w = w.at[0:40, 1 * P:2 * P].set(tap(W2, 0) * f2)
    w = w.at[0:40, 2 * P:3 * P].set(tap(W4, 0) * f4)
    w = w.at[0:40, 3 * P:4 * P].set(tap(W4, 1) * f4)
    w = w.at[nc:nc + 40, 0 * P:1 * P].set(tap(W1, 1) * f1)
    w = w.at[nc + 40, 0 * P:1 * P].set(_NEG)
    w = w.at[2 * nc:2 * nc + 40, 1 * P:2 * P].set(tap(W2, 1) * f2)
    w = w.at[2 * nc + 40, 1 * P:2 * P].set(_NEG)
    w = w.at[3 * nc, 0 * P:1 * P].set(b1 * f1)
    w = w.at[3 * nc, 1 * P:2 * P].set(b2 * f2)
    w = w.at[3 * nc, 2 * P:3 * P].set(b4 * f4)
    w = w.at[3 * nc + 1, 2 * P:3 * P].set(_NEG)
    w = w.astype(jnp.bfloat16)

    idx = discarded_idx.astype(jnp.int32)
    edge = jnp.full((B, 1), 40, jnp.int32)
    cols = [idx]
    for off in (1, 2):
        cols.append(jnp.concatenate(
            [idx[:, off:], jnp.broadcast_to(edge, (B, off))], axis=1))
    t = jnp.arange(N, dtype=jnp.int32)
    cols.append(jnp.broadcast_to((t >= N - 4).astype(jnp.int32), (B, N)))
    idx4 = jnp.stack(cols, axis=-1).reshape(B * N, 4)

    tb = block_boards
    body = functools.partial(_pattern_kernel, n_rows=N)
    out = pl.pallas_call(
        body,
        out_shape=jax.ShapeDtypeStruct((B, P), jnp.float32),
        grid=(B // tb,),
        in_specs=[
            pl.BlockSpec((tb * N, 4), lambda i: (i, 0)),
            pl.BlockSpec((_KW, 4 * P), lambda i: (0, 0)),
        ],
        out_specs=pl.BlockSpec((tb, P), lambda i: (i, 0)),
        compiler_params=pltpu.CompilerParams(
            dimension_semantics=("parallel",)),
    )(idx4, w)
    return out


def kernel(discarded_idx, card_features, W1, b1, W2, b2, W4, b4):
    return _pattern_module(discarded_idx, card_features,
                           W1, b1, W2, b2, W4, b4)
